# speedup vs baseline: 8.2273x; 8.2273x over previous
import functools

import jax
import jax.numpy as jnp
from jax import lax
from jax.experimental import pallas as pl
from jax.experimental.pallas import tpu as pltpu
from jax.experimental.pallas import tpu_sc as plsc

N = 10000
D = 128
E = 320000
NC = 2
NS = 16
NW = NC * NS
K = 128
EPW = 10240
NB = EPW // K
E_PAD = NW * EPW
ACC_N = 10240
ZROWS = ACC_N // NW

_mesh = plsc.VectorSubcoreMesh(core_axis_name="c", subcore_axis_name="s")


@functools.partial(
    pl.kernel,
    out_type=jax.ShapeDtypeStruct((NC, ACC_N, 16), jnp.float32),
    mesh=_mesh,
    scratch_types=[
        pltpu.VMEM((NB, K), jnp.int32),
        pltpu.VMEM((K, 16), jnp.float32),
        pltpu.VMEM_SHARED((ACC_N, 16), jnp.float32),
    ],
)
def _sc_degree(dst_hbm, ones_hbm, zeros_hbm, out_hbm, dst_v, ones_v, acc_sh):
    c = lax.axis_index("c")
    s = lax.axis_index("s")
    wid = c * NS + s
    pltpu.sync_copy(dst_hbm.at[wid], dst_v)
    pltpu.sync_copy(ones_hbm, ones_v)
    pltpu.sync_copy(zeros_hbm, acc_sh.at[pl.ds(s * ZROWS, ZROWS)])
    plsc.subcore_barrier()

    @pl.loop(0, NB)
    def _(b):
        pltpu.sync_copy(ones_v, acc_sh.at[dst_v.at[b]], add=True)

    plsc.subcore_barrier()
    pltpu.sync_copy(
        acc_sh.at[pl.ds(s * ZROWS, ZROWS)],
        out_hbm.at[c, pl.ds(s * ZROWS, ZROWS)],
    )


@functools.partial(
    pl.kernel,
    out_type=jax.ShapeDtypeStruct((NC, ACC_N, D), jnp.float32),
    mesh=_mesh,
    scratch_types=[
        pltpu.VMEM((NB, K), jnp.int32),
        pltpu.VMEM((NB, K), jnp.int32),
        pltpu.VMEM((K, D), jnp.float32),
        pltpu.VMEM_SHARED((ACC_N, D), jnp.float32),
    ],
)
def _sc_edge(h_hbm, src_hbm, dst_hbm, zeros_hbm, out_hbm,
             src_v, dst_v, rows_v, acc_sh):
    c = lax.axis_index("c")
    s = lax.axis_index("s")
    wid = c * NS + s
    pltpu.sync_copy(src_hbm.at[wid], src_v)
    pltpu.sync_copy(dst_hbm.at[wid], dst_v)
    pltpu.sync_copy(zeros_hbm, acc_sh.at[pl.ds(s * ZROWS, ZROWS)])
    plsc.subcore_barrier()

    @pl.loop(0, NB)
    def _(b):
        pltpu.sync_copy(h_hbm.at[src_v.at[b]], rows_v)
        pltpu.sync_copy(rows_v, acc_sh.at[dst_v.at[b]], add=True)

    plsc.subcore_barrier()
    pltpu.sync_copy(
        acc_sh.at[pl.ds(s * ZROWS, ZROWS)],
        out_hbm.at[c, pl.ds(s * ZROWS, ZROWS)],
    )


def _dis(deg_ref):
    deg = deg_ref[0, :N, 0:1] + deg_ref[1, :N, 0:1] + 1.0
    return lax.rsqrt(deg)


def _tc_prep_body(x_ref, w_ref, deg_ref, h1p_ref):
    dis = _dis(deg_ref)
    h = lax.dot_general(x_ref[...], w_ref[...], (((1,), (0,)), ((), ())),
                        precision=lax.Precision.HIGHEST,
                        preferred_element_type=jnp.float32)
    h1p_ref[...] = h * dis


def _tc_mid_body(acc_ref, h1p_ref, deg_ref, b1_ref, w2_ref, h2p_ref):
    dis = _dis(deg_ref)
    tot = acc_ref[0, :N, :] + acc_ref[1, :N, :] + h1p_ref[...]
    z1 = jnp.maximum(tot * dis + b1_ref[...], 0.0)
    h2 = lax.dot_general(z1, w2_ref[...], (((1,), (0,)), ((), ())),
                         precision=lax.Precision.HIGHEST,
                         preferred_element_type=jnp.float32)
    h2p_ref[...] = h2 * dis


def _tc_final_body(acc_ref, h2p_ref, deg_ref, b2_ref, out_ref):
    dis = _dis(deg_ref)
    tot = acc_ref[0, :N, :] + acc_ref[1, :N, :] + h2p_ref[...]
    out_ref[...] = tot * dis + b2_ref[...]


_tc_prep = pl.pallas_call(
    _tc_prep_body,
    out_shape=jax.ShapeDtypeStruct((N, D), jnp.float32),
)

_tc_mid = pl.pallas_call(
    _tc_mid_body,
    out_shape=jax.ShapeDtypeStruct((N, D), jnp.float32),
)

_tc_final = pl.pallas_call(
    _tc_final_body,
    out_shape=jax.ShapeDtypeStruct((N, D), jnp.float32),
)


@jax.jit
def kernel(x, edge_index, W1, b1, W2, b2):
    src = edge_index[0].astype(jnp.int32)
    dst = edge_index[1].astype(jnp.int32)
    pad = E_PAD - E
    src_p = jnp.concatenate([src, jnp.zeros((pad,), jnp.int32)])
    dst_p = jnp.concatenate(
        [dst, N + (jnp.arange(pad, dtype=jnp.int32) % (ACC_N - N))])
    src_g = src_p.reshape(NW, NB, K)
    dst_g = dst_p.reshape(NW, NB, K)

    zeros_d = jnp.zeros((ZROWS, D), jnp.float32)
    zeros_16 = jnp.zeros((ZROWS, 16), jnp.float32)
    ones_16 = jnp.ones((K, 16), jnp.float32)
    b1r = b1.reshape(1, D)
    b2r = b2.reshape(1, D)

    deg2 = _sc_degree(dst_g, ones_16, zeros_16)
    h1p = _tc_prep(x, W1, deg2)
    acc1 = _sc_edge(h1p, src_g, dst_g, zeros_d)
    h2p = _tc_mid(acc1, h1p, deg2, b1r, W2)
    acc2 = _sc_edge(h2p, src_g, dst_g, zeros_d)
    return _tc_final(acc2, h2p, deg2, b2r)
